# baseline (speedup 1.0000x reference)
import functools

import jax
import jax.numpy as jnp
from jax import lax
from jax.experimental import pallas as pl
from jax.experimental.pallas import tpu as pltpu
from jax.experimental.pallas import tpu_sc as plsc

_BATCH = 1024
_TOKENS = 20
_HIDDEN = 2048
_N = _BATCH * _TOKENS

_NC = 2
_NS = 16
_NW = _NC * _NS
_ROWS_PER_W = _N // _NW
_CHUNK = 32
_NCHUNK = _ROWS_PER_W // _CHUNK


def _gather_rows(weight, idx_flat):
    mesh = plsc.VectorSubcoreMesh(
        core_axis_name="core", subcore_axis_name="subcore"
    )

    @functools.partial(
        pl.kernel,
        out_type=jax.ShapeDtypeStruct((_N, _HIDDEN), weight.dtype),
        mesh=mesh,
        scratch_types=[
            pltpu.VMEM((_ROWS_PER_W,), jnp.int32),
            pltpu.VMEM((_CHUNK, _HIDDEN), jnp.float32),
        ],
    )
    def gather_kernel(w_hbm, i_hbm, o_hbm, idx_v, buf_v):
        wid = lax.axis_index("subcore") * _NC + lax.axis_index("core")
        base = wid * _ROWS_PER_W
        pltpu.sync_copy(i_hbm.at[pl.ds(base, _ROWS_PER_W)], idx_v)

        @pl.loop(0, _NCHUNK)
        def _(c):
            off = c * _CHUNK
            pltpu.sync_copy(w_hbm.at[idx_v.at[pl.ds(off, _CHUNK)]], buf_v)
            pltpu.sync_copy(buf_v, o_hbm.at[pl.ds(base + off, _CHUNK)])

    return gather_kernel(weight, idx_flat)


def kernel(indices, weight):
    idx_flat = indices.reshape(_N)
    out = _gather_rows(weight, idx_flat)
    return out.reshape(_BATCH, _TOKENS, _HIDDEN)
